# Initial kernel scaffold; baseline (speedup 1.0000x reference)
import jax
import jax.numpy as jnp
from jax.experimental import pallas as pl


def kernel(x, edge_index, W_disc, b_disc):
    raise NotImplementedError("write your pallas kernel here")


# speedup vs baseline: 3.9989x; 3.9989x over previous
import jax
import jax.numpy as jnp
from jax import lax
from jax.experimental import pallas as pl
from jax.experimental.pallas import tpu as pltpu
from jax.experimental.pallas import tpu_sc as plsc

N_NODES = 10000
N_EDGES = 320000
D = 128

NC = 2
NS = 16
NW = NC * NS

CH = 128
NCHUNK = 80
EDGES_PER_W = NCHUNK * CH
ACC_ROWS = 10240
ROWS_PER_TILE = ACC_ROWS // NS


def _mm_body(x_ref, w_ref, b_ref, h_ref):
    h_ref[...] = (
        jnp.dot(x_ref[...], w_ref[...], preferred_element_type=jnp.float32)
        + b_ref[...]
    )


def _combine_body(p_ref, o_ref):
    o_ref[...] = jnp.maximum(p_ref[0, :N_NODES] + p_ref[1, :N_NODES], 0.0)


SUP = 8


def _sc_agg_body(h_hbm, src_hbm, dst_hbm, out_hbm,
                 acc, src_idx, dst_idx, buf0, buf1,
                 gsem0, gsem1, ssem0, ssem1):
    cid = lax.axis_index("c")
    sid = lax.axis_index("s")
    wid = sid * NC + cid

    zero = jnp.zeros((16,), jnp.float32)

    def _zrow(i, carry):
        for k in range(D // 16):
            buf0[i, pl.ds(k * 16, 16)] = zero
        return carry

    lax.fori_loop(0, CH, _zrow, 0)
    base = sid * ROWS_PER_TILE
    for t in range(ROWS_PER_TILE // CH):
        pltpu.sync_copy(buf0, acc.at[pl.ds(base + t * CH, CH)])
    plsc.subcore_barrier()

    def _super(i, carry):
        pltpu.sync_copy(src_hbm.at[wid, pl.ds(i * SUP, SUP)], src_idx)
        pltpu.sync_copy(dst_hbm.at[wid, pl.ds(i * SUP, SUP)], dst_idx)
        for p in range(SUP // 2):
            j0 = 2 * p
            j1 = j0 + 1
            g0 = pltpu.async_copy(h_hbm.at[src_idx.at[j0]], buf0, gsem0)
            g1 = pltpu.async_copy(h_hbm.at[src_idx.at[j1]], buf1, gsem1)
            g0.wait()
            s0 = pltpu.async_copy(buf0, acc.at[dst_idx.at[j0]], ssem0,
                                  add=True)
            g1.wait()
            s1 = pltpu.async_copy(buf1, acc.at[dst_idx.at[j1]], ssem1,
                                  add=True)
            s0.wait()
            s1.wait()
        return carry

    lax.fori_loop(0, NCHUNK // SUP, _super, 0)
    plsc.subcore_barrier()

    for t in range(ROWS_PER_TILE // CH):
        r = base + t * CH
        pltpu.sync_copy(acc.at[pl.ds(r, CH)], out_hbm.at[cid, pl.ds(r, CH)])


def kernel(x, edge_index, W_disc, b_disc):
    h = pl.pallas_call(
        _mm_body,
        out_shape=jax.ShapeDtypeStruct((N_NODES, D), jnp.float32),
    )(x, W_disc, b_disc.reshape(1, D))

    src = edge_index[0].reshape(NW, N_EDGES // NW)
    dst = edge_index[1].reshape(NW, N_EDGES // NW)
    pad = EDGES_PER_W - N_EDGES // NW
    src_p = jnp.concatenate(
        [src, jnp.zeros((NW, pad), jnp.int32)], axis=1
    ).reshape(NW, NCHUNK, CH)
    dst_p = jnp.concatenate(
        [dst, jnp.full((NW, pad), N_NODES, jnp.int32)], axis=1
    ).reshape(NW, NCHUNK, CH)

    mesh = plsc.VectorSubcoreMesh(
        core_axis_name="c", subcore_axis_name="s",
        num_cores=NC, num_subcores=NS,
    )
    partial = pl.kernel(
        _sc_agg_body,
        out_type=jax.ShapeDtypeStruct((NC, ACC_ROWS, D), jnp.float32),
        mesh=mesh,
        scratch_types=[
            pltpu.VMEM_SHARED((ACC_ROWS, D), jnp.float32),
            pltpu.VMEM((SUP, CH), jnp.int32),
            pltpu.VMEM((SUP, CH), jnp.int32),
            pltpu.VMEM((CH, D), jnp.float32),
            pltpu.VMEM((CH, D), jnp.float32),
            pltpu.SemaphoreType.DMA,
            pltpu.SemaphoreType.DMA,
            pltpu.SemaphoreType.DMA,
            pltpu.SemaphoreType.DMA,
        ],
    )(h, src_p, dst_p)

    out = pl.pallas_call(
        _combine_body,
        out_shape=jax.ShapeDtypeStruct((N_NODES, D), jnp.float32),
    )(partial)
    return out


# speedup vs baseline: 4.2495x; 1.0627x over previous
import jax
import jax.numpy as jnp
from jax import lax
from jax.experimental import pallas as pl
from jax.experimental.pallas import tpu as pltpu
from jax.experimental.pallas import tpu_sc as plsc

N_NODES = 10000
N_EDGES = 320000
D = 128

NC = 2
NS = 16
NW = NC * NS

CH = 64
NB = 4
NCHUNK = 160
EDGES_PER_W = NCHUNK * CH
ACC_ROWS = 10240
ROWS_PER_TILE = ACC_ROWS // NS


def _mm_body(x_ref, w_ref, b_ref, h_ref):
    h_ref[...] = (
        jnp.dot(x_ref[...], w_ref[...], preferred_element_type=jnp.float32)
        + b_ref[...]
    )


def _combine_body(p_ref, o_ref):
    o_ref[...] = jnp.maximum(p_ref[0, :N_NODES] + p_ref[1, :N_NODES], 0.0)


SUP = 16


def _sc_agg_body(h_hbm, src_hbm, dst_hbm, out_hbm,
                 acc, src_idx, dst_idx, bufs, gsems, ssems):
    cid = lax.axis_index("c")
    sid = lax.axis_index("s")
    wid = sid * NC + cid

    zero = jnp.zeros((16,), jnp.float32)

    def _zrow(i, carry):
        for k in range(D // 16):
            bufs[0][i, pl.ds(k * 16, 16)] = zero
        return carry

    lax.fori_loop(0, CH, _zrow, 0)
    base = sid * ROWS_PER_TILE
    for t in range(ROWS_PER_TILE // CH):
        pltpu.sync_copy(bufs[0], acc.at[pl.ds(base + t * CH, CH)])
    plsc.subcore_barrier()

    def _gather(j, b):
        return pltpu.async_copy(h_hbm.at[src_idx.at[j]], bufs[b], gsems[b])

    def _scatter(j, b):
        return pltpu.async_copy(bufs[b], acc.at[dst_idx.at[j]], ssems[b],
                                add=True)

    def _super(i, carry):
        pltpu.sync_copy(src_hbm.at[wid, pl.ds(i * SUP, SUP)], src_idx)
        pltpu.sync_copy(dst_hbm.at[wid, pl.ds(i * SUP, SUP)], dst_idx)
        gd = {0: _gather(0, 0), 1: _gather(1, 1)}
        sd = {}
        for c in range(SUP):
            nxt = c + 2
            if nxt < SUP:
                b2 = nxt % NB
                if c >= 2:
                    sd[b2].wait()
                gd[b2] = _gather(nxt, b2)
            b = c % NB
            gd[b].wait()
            sd[b] = _scatter(c, b)
        for c in range(SUP - NB, SUP):
            sd[c % NB].wait()
        return carry

    lax.fori_loop(0, NCHUNK // SUP, _super, 0)
    plsc.subcore_barrier()

    for t in range(ROWS_PER_TILE // CH):
        r = base + t * CH
        pltpu.sync_copy(acc.at[pl.ds(r, CH)], out_hbm.at[cid, pl.ds(r, CH)])


def kernel(x, edge_index, W_disc, b_disc):
    h = pl.pallas_call(
        _mm_body,
        out_shape=jax.ShapeDtypeStruct((N_NODES, D), jnp.float32),
    )(x, W_disc, b_disc.reshape(1, D))

    src = edge_index[0].reshape(NW, N_EDGES // NW)
    dst = edge_index[1].reshape(NW, N_EDGES // NW)
    pad = EDGES_PER_W - N_EDGES // NW
    src_p = jnp.concatenate(
        [src, jnp.zeros((NW, pad), jnp.int32)], axis=1
    ).reshape(NW, NCHUNK, CH)
    dst_p = jnp.concatenate(
        [dst, jnp.full((NW, pad), N_NODES, jnp.int32)], axis=1
    ).reshape(NW, NCHUNK, CH)

    mesh = plsc.VectorSubcoreMesh(
        core_axis_name="c", subcore_axis_name="s",
        num_cores=NC, num_subcores=NS,
    )
    partial = pl.kernel(
        _sc_agg_body,
        out_type=jax.ShapeDtypeStruct((NC, ACC_ROWS, D), jnp.float32),
        mesh=mesh,
        scratch_types=[
            pltpu.VMEM_SHARED((ACC_ROWS, D), jnp.float32),
            pltpu.VMEM((SUP, CH), jnp.int32),
            pltpu.VMEM((SUP, CH), jnp.int32),
            [pltpu.VMEM((CH, D), jnp.float32) for _ in range(NB)],
            [pltpu.SemaphoreType.DMA for _ in range(NB)],
            [pltpu.SemaphoreType.DMA for _ in range(NB)],
        ],
    )(h, src_p, dst_p)

    out = pl.pallas_call(
        _combine_body,
        out_shape=jax.ShapeDtypeStruct((N_NODES, D), jnp.float32),
    )(partial)
    return out
